# baseline (speedup 1.0000x reference)
import functools

import jax
import jax.numpy as jnp
from jax import lax
from jax.experimental import pallas as pl
from jax.experimental.pallas import tpu as pltpu
from jax.experimental.pallas import tpu_sc as plsc

HID = 64
SEQ = 200
CH = 128
NBUF = 4
LOOKAHEAD = 2
NG = CH // 16


def _rsqrt(v):
    i = plsc.bitcast(v, jnp.int32)
    y = plsc.bitcast(jnp.int32(0x5F3759DF) - (i >> 1), jnp.float32)
    for _ in range(3):
        y = y * (1.5 - 0.5 * v * y * y)
    return y


def _fused(ids3, r3, t3, table, pos_flat, rt_flat, st_flat, gamma, beta):
    nw, nchunk, _ = ids3.shape
    n = nw * nchunk * CH
    nr = 4
    ns = 10

    mesh = plsc.VectorSubcoreMesh(core_axis_name="c", subcore_axis_name="s")

    @functools.partial(
        pl.kernel,
        mesh=mesh,
        compiler_params=pltpu.CompilerParams(use_tc_tiling_on_sc=False,
                                             needs_layout_passes=False),
        out_type=jax.ShapeDtypeStruct((n, HID), jnp.float32),
        scratch_types=(
            [pltpu.VMEM((nchunk, CH), jnp.int32),
             pltpu.VMEM((SEQ * HID,), jnp.float32),
             pltpu.VMEM((nr * HID,), jnp.float32),
             pltpu.VMEM((ns * HID,), jnp.float32),
             pltpu.VMEM((nr * ns * HID,), jnp.float32),
             pltpu.VMEM((HID,), jnp.float32),
             pltpu.VMEM((HID,), jnp.float32)]
            + [pltpu.VMEM((CH, HID), jnp.float32) for _ in range(NBUF)]
            + [pltpu.VMEM((CH,), jnp.int32) for _ in range(2 * NBUF)]
            + [pltpu.SemaphoreType.DMA for _ in range(3 * NBUF)]
        ),
    )
    def fused_kernel(ids_hbm, r_hbm, t_hbm, table_hbm, pos_hbm, rt_hbm,
                     st_hbm, g_hbm, b_hbm, out_hbm, idx_v, pos_v, rt_v, st_v,
                     comb_v, g_v, b_v, *rest):
        bufs = rest[:NBUF]
        rbufs = rest[NBUF:2 * NBUF]
        tbufs = rest[2 * NBUF:3 * NBUF]
        gsem = rest[3 * NBUF:4 * NBUF]
        asem = rest[4 * NBUF:5 * NBUF]
        wsem = rest[5 * NBUF:]
        nc = plsc.get_sparse_core_info().num_cores
        wid = lax.axis_index("s") * nc + lax.axis_index("c")
        base = wid * (nchunk * CH)

        pltpu.sync_copy(ids_hbm.at[wid], idx_v)
        pltpu.sync_copy(pos_hbm, pos_v)
        pltpu.sync_copy(rt_hbm, rt_v)
        pltpu.sync_copy(st_hbm, st_v)
        pltpu.sync_copy(g_hbm, g_v)
        pltpu.sync_copy(b_hbm, b_v)

        for a in range(nr * ns):
            for k in range(0, HID, 16):
                comb_v[pl.ds(a * HID + k, 16)] = (
                    rt_v[pl.ds((a // ns) * HID + k, 16)]
                    + st_v[pl.ds((a % ns) * HID + k, 16)])

        iota = lax.broadcasted_iota(jnp.int32, (16,), 0)
        rows_g = [g * 16 + iota for g in range(NG)]

        def g_desc(j, b):
            return pltpu.make_async_copy(
                table_hbm.at[idx_v.at[j]], bufs[b], gsem[b])

        def a_descs(j, b):
            return (pltpu.make_async_copy(r_hbm.at[wid, j], rbufs[b],
                                          asem[b]),
                    pltpu.make_async_copy(t_hbm.at[wid, j], tbufs[b],
                                          asem[b]))

        def w_desc(j, b):
            return pltpu.make_async_copy(
                bufs[b], out_hbm.at[pl.ds(base + j * CH, CH)], wsem[b])

        def start_chunk(j, b):
            g_desc(j, b).start()
            da, db = a_descs(j, b)
            da.start()
            db.start()

        for p in range(LOOKAHEAD):
            start_chunk(p, p)

        def chunk_body(g, carry):
            for b in range(NBUF):
                j = g * NBUF + b
                g_desc(j, b).wait()
                da, db = a_descs(j, b)
                da.wait()
                db.wait()

                comb_base = []
                pos_base = []
                for gi in range(NG):
                    rv = rbufs[b][pl.ds(gi * 16, 16)]
                    tv = tbufs[b][pl.ds(gi * 16, 16)]
                    comb_base.append((rv * ns + tv) * HID)
                    srow = jnp.full((16,), base + j * CH + gi * 16,
                                    jnp.int32) + iota
                    pos_base.append(lax.rem(srow, jnp.int32(SEQ)) * HID)

                def pass1(h, c):
                    hs = c[0]
                    s1 = list(c[1])
                    s2 = list(c[2])
                    for gi in range(NG):
                        x = plsc.load_gather(bufs[b], [rows_g[gi], hs])
                        x = x + plsc.load_gather(comb_v, [comb_base[gi] + hs])
                        x = x + plsc.load_gather(pos_v, [pos_base[gi] + hs])
                        s1[gi] = s1[gi] + x
                        s2[gi] = s2[gi] + x * x
                        plsc.store_scatter(bufs[b], [rows_g[gi], hs], x)
                    return (hs + 1, tuple(s1), tuple(s2))

                zf = tuple(jnp.zeros((16,), jnp.float32) for _ in range(NG))
                hs0 = jnp.zeros((16,), jnp.int32)
                _, s1, s2 = lax.fori_loop(0, HID, pass1, (hs0, zf, zf))

                inv = jnp.float32(1.0 / HID)
                means = [s1[gi] * inv for gi in range(NG)]
                rstds = [_rsqrt(s2[gi] * inv - means[gi] * means[gi] + 1e-5)
                         for gi in range(NG)]

                def pass2(h, hs):
                    gs = plsc.load_gather(g_v, [hs])
                    bs = plsc.load_gather(b_v, [hs])
                    for gi in range(NG):
                        x = plsc.load_gather(bufs[b], [rows_g[gi], hs])
                        y = (x - means[gi]) * (rstds[gi] * gs) + bs
                        plsc.store_scatter(bufs[b], [rows_g[gi], hs], y)
                    return hs + 1

                lax.fori_loop(0, HID, pass2, hs0)

                w_desc(j, b).start()
                jn = j + LOOKAHEAD
                bn = (b + LOOKAHEAD) % NBUF

                @pl.when(jn < nchunk)
                def _():
                    @pl.when(jn >= NBUF)
                    def _():
                        w_desc(jn - NBUF, bn).wait()
                    start_chunk(jn, bn)
            return carry

        lax.fori_loop(0, nchunk // NBUF, chunk_body, 0)

        for b in range(NBUF):
            w_desc(nchunk - NBUF + b, b).wait()

    return fused_kernel(ids3, r3, t3, table, pos_flat, rt_flat, st_flat,
                        gamma, beta)


def kernel(input_ids, reasoning_ids, step_positions, token_table, pos_table,
           reasoning_table, step_table, ln_gamma, ln_beta):
    b, s = input_ids.shape
    info = plsc.get_sparse_core_info()
    nw = info.num_cores * info.num_subcores
    n = b * s
    nchunk = n // (nw * CH)
    ids3 = input_ids.astype(jnp.int32).reshape(nw, nchunk, CH)
    r3 = reasoning_ids.astype(jnp.int32).reshape(nw, nchunk, CH)
    t3 = step_positions.astype(jnp.int32).reshape(nw, nchunk, CH)
    pos_flat = lax.slice_in_dim(pos_table, 0, s, axis=0).reshape(-1)
    out = _fused(ids3, r3, t3, token_table, pos_flat,
                 reasoning_table.reshape(-1), step_table.reshape(-1),
                 ln_gamma, ln_beta)
    return out.reshape(b, s, HID)


# speedup vs baseline: 5.6870x; 5.6870x over previous
import functools

import jax
import jax.numpy as jnp
from jax import lax
from jax.experimental import pallas as pl
from jax.experimental.pallas import tpu as pltpu
from jax.experimental.pallas import tpu_sc as plsc

HID = 64
SEQ = 200
CH = 128
NBUF = 4
NG = CH // 16


def _rsqrt(v):
    i = plsc.bitcast(v, jnp.int32)
    y = plsc.bitcast(jnp.int32(0x5F3759DF) - (i >> 1), jnp.float32)
    for _ in range(3):
        y = y * (1.5 - 0.5 * v * y * y)
    return y


def _fused(ids3h, par3, r3, t3, tabv, pos_flat, rt_flat, st_flat, gamma,
           beta):
    nw, nchunk, _ = ids3h.shape
    n = nw * nchunk * CH
    nr = 4
    ns = 10

    mesh = plsc.VectorSubcoreMesh(core_axis_name="c", subcore_axis_name="s")

    @functools.partial(
        pl.kernel,
        mesh=mesh,
        compiler_params=pltpu.CompilerParams(needs_layout_passes=False),
        out_type=jax.ShapeDtypeStruct((n, HID), jnp.float32),
        scratch_types=(
            [pltpu.VMEM((SEQ * HID,), jnp.float32),
             pltpu.VMEM((nr * HID,), jnp.float32),
             pltpu.VMEM((ns * HID,), jnp.float32),
             pltpu.VMEM((nr * ns * HID,), jnp.float32),
             pltpu.VMEM((HID,), jnp.float32),
             pltpu.VMEM((HID,), jnp.float32),
             pltpu.VMEM((2 * CH,), jnp.int32)]
            + [pltpu.VMEM((CH, 2 * HID), jnp.float32) for _ in range(NBUF)]
            + [pltpu.VMEM((CH, HID), jnp.float32) for _ in range(2)]
            + [pltpu.VMEM((CH,), jnp.int32) for _ in range(4 * NBUF)]
            + [pltpu.SemaphoreType.DMA for _ in range(2 * NBUF + 2)]
        ),
    )
    def fused_kernel(ids_hbm, par_hbm, r_hbm, t_hbm, table_hbm, pos_hbm,
                     rt_hbm, st_hbm, g_hbm, b_hbm, out_hbm, pos_v, rt_v,
                     st_v, comb_v, g_v, b_v, prep_v, *rest):
        bufs = rest[:NBUF]
        ybufs = rest[NBUF:NBUF + 2]
        ibufs = rest[NBUF + 2:2 * NBUF + 2]
        rbufs = rest[2 * NBUF + 2:3 * NBUF + 2]
        tbufs = rest[3 * NBUF + 2:4 * NBUF + 2]
        pbufs = rest[4 * NBUF + 2:5 * NBUF + 2]
        gsem = rest[5 * NBUF + 2:6 * NBUF + 2]
        asem = rest[6 * NBUF + 2:7 * NBUF + 2]
        wsem = rest[7 * NBUF + 2:]
        nc = plsc.get_sparse_core_info().num_cores
        wid = lax.axis_index("s") * nc + lax.axis_index("c")
        base = wid * (nchunk * CH)

        pltpu.sync_copy(pos_hbm, pos_v)
        pltpu.sync_copy(rt_hbm, rt_v)
        pltpu.sync_copy(st_hbm, st_v)
        pltpu.sync_copy(g_hbm, g_v)
        pltpu.sync_copy(b_hbm, b_v)

        for a in range(nr * ns):
            for k in range(0, HID, 16):
                comb_v[pl.ds(a * HID + k, 16)] = (
                    rt_v[pl.ds((a // ns) * HID + k, 16)]
                    + st_v[pl.ds((a % ns) * HID + k, 16)])

        iota = lax.broadcasted_iota(jnp.int32, (16,), 0)
        rows_g = [g * 16 + iota for g in range(NG)]
        hg = NG // 2

        def aux_descs(j, ab):
            return [
                pltpu.make_async_copy(ids_hbm.at[wid, j], ibufs[ab],
                                      asem[ab]),
                pltpu.make_async_copy(par_hbm.at[wid, j], pbufs[ab],
                                      asem[ab]),
                pltpu.make_async_copy(r_hbm.at[wid, j], rbufs[ab], asem[ab]),
                pltpu.make_async_copy(t_hbm.at[wid, j], tbufs[ab], asem[ab]),
            ]

        def g_desc(j, b):
            return pltpu.make_async_copy(
                table_hbm.at[ibufs[b]], bufs[b], gsem[b])

        def w_desc(j, yb):
            return pltpu.make_async_copy(
                ybufs[yb], out_hbm.at[pl.ds(base + j * CH, CH)], wsem[yb])

        for d in aux_descs(0, 0):
            d.start()
        for d in aux_descs(1, 1):
            d.start()
        for d in aux_descs(0, 0):
            d.wait()
        g_desc(0, 0).start()

        def chunk_body(g, carry):
            for b in range(NBUF):
                j = g * NBUF + b
                jn = j + 1
                bn = (b + 1) % NBUF
                jn2 = j + 2
                bn2 = (b + 2) % NBUF

                @pl.when(jn < nchunk)
                def _():
                    for d in aux_descs(jn, bn):
                        d.wait()
                    g_desc(jn, bn).start()

                @pl.when(jn2 < nchunk)
                def _():
                    for d in aux_descs(jn2, bn2):
                        d.start()

                g_desc(j, b).wait()

                @pl.when(j >= 2)
                def _():
                    w_desc(j - 2, b % 2).wait()

                for gi in range(NG):
                    rv = rbufs[b][pl.ds(gi * 16, 16)]
                    tv = tbufs[b][pl.ds(gi * 16, 16)]
                    prep_v[pl.ds(gi * 16, 16)] = (rv * ns + tv) * HID
                    srow = jnp.full((16,), base + j * CH + gi * 16,
                                    jnp.int32) + iota
                    prep_v[pl.ds(CH + gi * 16, 16)] = (
                        lax.rem(srow, jnp.int32(SEQ)) * HID)

                zf4 = tuple(jnp.zeros((16,), jnp.float32) for _ in range(hg))
                means = []
                rstds = []
                inv = jnp.float32(1.0 / HID)
                for half in range(2):
                    g0 = half * hg

                    @plsc.parallel_loop(0, HID, unroll=2, carry=(zf4, zf4))
                    def pass1_out(h, c):
                        cs = (jnp.full((16,), h, jnp.int32) + iota) & (HID - 1)
                        s1 = list(c[0])
                        s2 = list(c[1])
                        xs = []
                        tcs = []
                        for k in range(hg):
                            gi = g0 + k
                            ci = prep_v[pl.ds(gi * 16, 16)] + cs
                            pi = prep_v[pl.ds(CH + gi * 16, 16)] + cs
                            tc = pbufs[b][pl.ds(gi * 16, 16)] + cs
                            x = plsc.load_gather(bufs[b], [rows_g[gi], tc])
                            x = x + plsc.load_gather(comb_v, [ci])
                            x = x + plsc.load_gather(pos_v, [pi])
                            s1[k] = s1[k] + x
                            s2[k] = s2[k] + x * x
                            xs.append(x)
                            tcs.append(tc)
                        for k in range(hg):
                            plsc.store_scatter(bufs[b], [rows_g[g0 + k],
                                                         tcs[k]], xs[k])
                        return (tuple(s1), tuple(s2))

                    s1, s2 = pass1_out
                    for k in range(hg):
                        m = s1[k] * inv
                        means.append(m)
                        rstds.append(_rsqrt(s2[k] * inv - m * m + 1e-5))

                for half in range(2):
                    g0 = half * hg

                    @plsc.parallel_loop(0, HID, unroll=2, carry=jnp.int32(0))
                    def pass2_out(h, cdummy):
                        cs = (jnp.full((16,), h, jnp.int32) + iota) & (HID - 1)
                        gs = plsc.load_gather(g_v, [cs])
                        bs = plsc.load_gather(b_v, [cs])
                        ys = []
                        for k in range(hg):
                            gi = g0 + k
                            tc = pbufs[b][pl.ds(gi * 16, 16)] + cs
                            x = plsc.load_gather(bufs[b], [rows_g[gi], tc])
                            ys.append((x - means[gi])
                                      * (rstds[gi] * gs) + bs)
                        for k in range(hg):
                            plsc.store_scatter(
                                ybufs[b % 2], [rows_g[g0 + k], cs], ys[k])
                        return cdummy

                    del pass2_out

                w_desc(j, b % 2).start()
            return carry

        lax.fori_loop(0, nchunk // NBUF, chunk_body, 0)

        for k in range(2):
            w_desc(nchunk - 2 + k, (nchunk - 2 + k) % 2).wait()

    return fused_kernel(ids3h, par3, r3, t3, tabv, pos_flat, rt_flat,
                        st_flat, gamma, beta)


def kernel(input_ids, reasoning_ids, step_positions, token_table, pos_table,
           reasoning_table, step_table, ln_gamma, ln_beta):
    b, s = input_ids.shape
    info = plsc.get_sparse_core_info()
    nw = info.num_cores * info.num_subcores
    n = b * s
    nchunk = n // (nw * CH)
    ids = input_ids.astype(jnp.int32)
    ids3h = (ids >> 1).reshape(nw, nchunk, CH)
    par3 = ((ids & 1) * HID).reshape(nw, nchunk, CH)
    r3 = reasoning_ids.astype(jnp.int32).reshape(nw, nchunk, CH)
    t3 = step_positions.astype(jnp.int32).reshape(nw, nchunk, CH)
    tabv = token_table.reshape(token_table.shape[0] // 2, 2 * HID)
    pos_flat = lax.slice_in_dim(pos_table, 0, s, axis=0).reshape(-1)
    out = _fused(ids3h, par3, r3, t3, tabv, pos_flat,
                 reasoning_table.reshape(-1), step_table.reshape(-1),
                 ln_gamma, ln_beta)
    return out.reshape(b, s, HID)


# speedup vs baseline: 5.8411x; 1.0271x over previous
import functools

import jax
import jax.numpy as jnp
from jax import lax
from jax.experimental import pallas as pl
from jax.experimental.pallas import tpu as pltpu
from jax.experimental.pallas import tpu_sc as plsc

HID = 64
SEQ = 200
CH = 128
NBUF = 4
NG = CH // 16


def _rsqrt(v):
    i = plsc.bitcast(v, jnp.int32)
    y = plsc.bitcast(jnp.int32(0x5F3759DF) - (i >> 1), jnp.float32)
    for _ in range(3):
        y = y * (1.5 - 0.5 * v * y * y)
    return y


def _fused(ids3h, par3, r3, t3, tabv, pos_flat, rt_flat, st_flat, gamma,
           beta):
    nw, nchunk, _ = ids3h.shape
    n = nw * nchunk * CH
    nr = 4
    ns = 10

    mesh = plsc.VectorSubcoreMesh(core_axis_name="c", subcore_axis_name="s")

    @functools.partial(
        pl.kernel,
        mesh=mesh,
        compiler_params=pltpu.CompilerParams(needs_layout_passes=False),
        out_type=jax.ShapeDtypeStruct((n, HID), jnp.float32),
        scratch_types=(
            [pltpu.VMEM((SEQ * HID,), jnp.float32),
             pltpu.VMEM((nr * HID,), jnp.float32),
             pltpu.VMEM((ns * HID,), jnp.float32),
             pltpu.VMEM((nr * ns * HID,), jnp.float32),
             pltpu.VMEM((HID,), jnp.float32),
             pltpu.VMEM((HID,), jnp.float32),
             pltpu.VMEM((2 * CH,), jnp.int32)]
            + [pltpu.VMEM((CH, 2 * HID), jnp.float32) for _ in range(NBUF)]
            + [pltpu.VMEM((CH, HID), jnp.float32) for _ in range(2)]
            + [pltpu.VMEM((CH,), jnp.int32) for _ in range(4 * NBUF)]
            + [pltpu.SemaphoreType.DMA for _ in range(2 * NBUF + 2)]
        ),
    )
    def fused_kernel(ids_hbm, par_hbm, r_hbm, t_hbm, table_hbm, pos_hbm,
                     rt_hbm, st_hbm, g_hbm, b_hbm, out_hbm, pos_v, rt_v,
                     st_v, comb_v, g_v, b_v, prep_v, *rest):
        bufs = rest[:NBUF]
        ybufs = rest[NBUF:NBUF + 2]
        ibufs = rest[NBUF + 2:2 * NBUF + 2]
        rbufs = rest[2 * NBUF + 2:3 * NBUF + 2]
        tbufs = rest[3 * NBUF + 2:4 * NBUF + 2]
        pbufs = rest[4 * NBUF + 2:5 * NBUF + 2]
        gsem = rest[5 * NBUF + 2:6 * NBUF + 2]
        asem = rest[6 * NBUF + 2:7 * NBUF + 2]
        wsem = rest[7 * NBUF + 2:]
        nc = plsc.get_sparse_core_info().num_cores
        wid = lax.axis_index("s") * nc + lax.axis_index("c")
        base = wid * (nchunk * CH)

        pltpu.sync_copy(pos_hbm, pos_v)
        pltpu.sync_copy(rt_hbm, rt_v)
        pltpu.sync_copy(st_hbm, st_v)
        pltpu.sync_copy(g_hbm, g_v)
        pltpu.sync_copy(b_hbm, b_v)

        for a in range(nr * ns):
            for k in range(0, HID, 16):
                comb_v[pl.ds(a * HID + k, 16)] = (
                    rt_v[pl.ds((a // ns) * HID + k, 16)]
                    + st_v[pl.ds((a % ns) * HID + k, 16)])

        iota = lax.broadcasted_iota(jnp.int32, (16,), 0)
        rows_g = [g * 16 + iota for g in range(NG)]
        hg = NG // 2

        def aux_descs(j, ab):
            return [
                pltpu.make_async_copy(ids_hbm.at[wid, j], ibufs[ab],
                                      asem[ab]),
                pltpu.make_async_copy(par_hbm.at[wid, j], pbufs[ab],
                                      asem[ab]),
                pltpu.make_async_copy(r_hbm.at[wid, j], rbufs[ab], asem[ab]),
                pltpu.make_async_copy(t_hbm.at[wid, j], tbufs[ab], asem[ab]),
            ]

        def g_desc(j, b):
            return pltpu.make_async_copy(
                table_hbm.at[ibufs[b]], bufs[b], gsem[b])

        def w_desc(j, yb):
            return pltpu.make_async_copy(
                ybufs[yb], out_hbm.at[pl.ds(base + j * CH, CH)], wsem[yb])

        for d in aux_descs(0, 0):
            d.start()
        for d in aux_descs(1, 1):
            d.start()
        for d in aux_descs(0, 0):
            d.wait()
        g_desc(0, 0).start()

        def chunk_body(g, carry):
            for b in range(NBUF):
                j = g * NBUF + b
                jn = j + 1
                bn = (b + 1) % NBUF
                jn2 = j + 2
                bn2 = (b + 2) % NBUF

                @pl.when(jn < nchunk)
                def _():
                    for d in aux_descs(jn, bn):
                        d.wait()
                    g_desc(jn, bn).start()

                @pl.when(jn2 < nchunk)
                def _():
                    for d in aux_descs(jn2, bn2):
                        d.start()

                g_desc(j, b).wait()

                @pl.when(j >= 2)
                def _():
                    w_desc(j - 2, b % 2).wait()

                for gi in range(NG):
                    rv = rbufs[b][pl.ds(gi * 16, 16)]
                    tv = tbufs[b][pl.ds(gi * 16, 16)]
                    prep_v[pl.ds(gi * 16, 16)] = (rv * ns + tv) * HID
                    srow = jnp.full((16,), base + j * CH + gi * 16,
                                    jnp.int32) + iota
                    prep_v[pl.ds(CH + gi * 16, 16)] = (
                        lax.rem(srow, jnp.int32(SEQ)) * HID)

                zf4 = tuple(jnp.zeros((16,), jnp.float32) for _ in range(hg))
                means = []
                rstds = []
                inv = jnp.float32(1.0 / HID)
                for half in range(2):
                    g0 = half * hg

                    @plsc.parallel_loop(0, HID, unroll=2, carry=(zf4, zf4))
                    def pass1_out(h, c):
                        cs = (jnp.full((16,), h, jnp.int32) + iota) & (HID - 1)
                        s1 = list(c[0])
                        s2 = list(c[1])
                        xs = []
                        tcs = []
                        for k in range(hg):
                            gi = g0 + k
                            ci = prep_v[pl.ds(gi * 16, 16)] + cs
                            pi = prep_v[pl.ds(CH + gi * 16, 16)] + cs
                            tc = pbufs[b][pl.ds(gi * 16, 16)] + cs
                            x = plsc.load_gather(bufs[b], [rows_g[gi], tc])
                            x = x + plsc.load_gather(comb_v, [ci])
                            x = x + plsc.load_gather(pos_v, [pi])
                            s1[k] = s1[k] + x
                            s2[k] = s2[k] + x * x
                            xs.append(x)
                        for k in range(hg):
                            plsc.store_scatter(bufs[b], [rows_g[g0 + k], cs],
                                               xs[k])
                        return (tuple(s1), tuple(s2))

                    s1, s2 = pass1_out
                    for k in range(hg):
                        m = s1[k] * inv
                        means.append(m)
                        rstds.append(_rsqrt(s2[k] * inv - m * m + 1e-5))

                for half in range(2):
                    g0 = half * hg

                    @plsc.parallel_loop(0, HID, unroll=2, carry=jnp.int32(0))
                    def pass2_out(h, cdummy):
                        cs = (jnp.full((16,), h, jnp.int32) + iota) & (HID - 1)
                        gs = plsc.load_gather(g_v, [cs])
                        bs = plsc.load_gather(b_v, [cs])
                        ys = []
                        for k in range(hg):
                            gi = g0 + k
                            x = plsc.load_gather(bufs[b], [rows_g[gi], cs])
                            ys.append((x - means[gi])
                                      * (rstds[gi] * gs) + bs)
                        for k in range(hg):
                            plsc.store_scatter(
                                ybufs[b % 2], [rows_g[g0 + k], cs], ys[k])
                        return cdummy

                    del pass2_out

                w_desc(j, b % 2).start()
            return carry

        lax.fori_loop(0, nchunk // NBUF, chunk_body, 0)

        for k in range(2):
            w_desc(nchunk - 2 + k, (nchunk - 2 + k) % 2).wait()

    return fused_kernel(ids3h, par3, r3, t3, tabv, pos_flat, rt_flat,
                        st_flat, gamma, beta)


def kernel(input_ids, reasoning_ids, step_positions, token_table, pos_table,
           reasoning_table, step_table, ln_gamma, ln_beta):
    b, s = input_ids.shape
    info = plsc.get_sparse_core_info()
    nw = info.num_cores * info.num_subcores
    n = b * s
    nchunk = n // (nw * CH)
    ids = input_ids.astype(jnp.int32)
    ids3h = (ids >> 1).reshape(nw, nchunk, CH)
    par3 = ((ids & 1) * HID).reshape(nw, nchunk, CH)
    r3 = reasoning_ids.astype(jnp.int32).reshape(nw, nchunk, CH)
    t3 = step_positions.astype(jnp.int32).reshape(nw, nchunk, CH)
    tabv = token_table.reshape(token_table.shape[0] // 2, 2 * HID)
    pos_flat = lax.slice_in_dim(pos_table, 0, s, axis=0).reshape(-1)
    out = _fused(ids3h, par3, r3, t3, tabv, pos_flat,
                 reasoning_table.reshape(-1), step_table.reshape(-1),
                 ln_gamma, ln_beta)
    return out.reshape(b, s, HID)
